# Initial kernel scaffold; baseline (speedup 1.0000x reference)
import jax
import jax.numpy as jnp
from jax.experimental import pallas as pl


def kernel(x, edge_index, edge_attr, W_in, b_in, edge_W, edge_b, gru_W_ih, gru_W_hh, gru_b_ih, gru_b_hh, W_t1, b_t1, W_t2, b_t2):
    raise NotImplementedError("write your pallas kernel here")


# speedup vs baseline: 5.4203x; 5.4203x over previous
import functools

import jax
import jax.numpy as jnp
from jax import lax
from jax.experimental import pallas as pl
from jax.experimental.pallas import tpu as pltpu
from jax.experimental.pallas import tpu_sc as plsc

_STEPS = 4
_HIGH = jax.lax.Precision.HIGHEST


def _hinit_body(x_ref, w_ref, b_ref, o_ref):
    o_ref[...] = (
        jnp.dot(x_ref[...], w_ref[...], precision=_HIGH) + b_ref[...]
    )


_RB = 2000


def _hinit(x, w, b):
    n, d = x.shape
    h = w.shape[1]
    return pl.pallas_call(
        _hinit_body,
        grid=(n // _RB,),
        in_specs=[
            pl.BlockSpec((_RB, d), lambda i: (i, 0)),
            pl.BlockSpec((d, h), lambda i: (0, 0)),
            pl.BlockSpec((1, h), lambda i: (0, 0)),
        ],
        out_specs=pl.BlockSpec((_RB, h), lambda i: (i, 0)),
        out_shape=jax.ShapeDtypeStruct((n, h), jnp.float32),
    )(x, w, b)


def _mtable_body(h_ref, w_ref, b_ref, o_ref):
    t = pl.program_id(0)
    m = jnp.dot(h_ref[...], w_ref[0], precision=_HIGH) + b_ref[t, 0]
    o_ref[0] = jnp.maximum(m, 0.0)


def _mtable(h, edge_w, edge_b):
    n, hd = h.shape
    t = edge_w.shape[0]
    return pl.pallas_call(
        _mtable_body,
        grid=(t, n // _RB),
        in_specs=[
            pl.BlockSpec((_RB, hd), lambda t, i: (i, 0)),
            pl.BlockSpec((1, hd, hd), lambda t, i: (t, 0, 0)),
            pl.BlockSpec(memory_space=pltpu.SMEM),
        ],
        out_specs=pl.BlockSpec((1, _RB, hd), lambda t, i: (t, i, 0)),
        out_shape=jax.ShapeDtypeStruct((t, n, hd), jnp.float32),
    )(h, edge_w, edge_b)


def _gru_body(agg_ref, h_ref, wir, wiz, win, whr, whz, whn, br, bz, bni, bnh,
              o_ref):
    agg = agg_ref[0] + agg_ref[1]
    h = h_ref[...]

    def dot(a, w_ref):
        return jnp.dot(a, w_ref[...], precision=_HIGH)

    r = jax.nn.sigmoid(dot(agg, wir) + dot(h, whr) + br[...])
    z = jax.nn.sigmoid(dot(agg, wiz) + dot(h, whz) + bz[...])
    n = jnp.tanh(dot(agg, win) + bni[...] + r * (dot(h, whn) + bnh[...]))
    o_ref[...] = (1.0 - z) * n + z * h


def _gru(agg2, h, ws, bs):
    n, hd = h.shape
    wspec = pl.BlockSpec((hd, hd), lambda i: (0, 0))
    bspec = pl.BlockSpec((1, hd), lambda i: (0, 0))
    return pl.pallas_call(
        _gru_body,
        grid=(n // _RB,),
        in_specs=[
            pl.BlockSpec((2, _RB, hd), lambda i: (0, i, 0)),
            pl.BlockSpec((_RB, hd), lambda i: (i, 0)),
        ] + [wspec] * 6 + [bspec] * 4,
        out_specs=pl.BlockSpec((_RB, hd), lambda i: (i, 0)),
        out_shape=jax.ShapeDtypeStruct((n, hd), jnp.float32),
    )(agg2, h, *ws, *bs)


def _final_body(hi_ref, h_ref, w1a, w1b, b1, w2, b2, o_ref):
    t1 = jnp.maximum(
        jnp.dot(hi_ref[...], w1a[...], precision=_HIGH)
        + jnp.dot(h_ref[...], w1b[...], precision=_HIGH)
        + b1[...],
        0.0,
    )
    o_ref[...] = jnp.tanh(jnp.dot(t1, w2[...], precision=_HIGH) + b2[...])


def _final(h_init, h, w1a, w1b, b1, w2, b2):
    n, hd = h.shape
    wspec = pl.BlockSpec((hd, hd), lambda i: (0, 0))
    bspec = pl.BlockSpec((1, hd), lambda i: (0, 0))
    return pl.pallas_call(
        _final_body,
        grid=(n // _RB,),
        in_specs=[
            pl.BlockSpec((_RB, hd), lambda i: (i, 0)),
            pl.BlockSpec((_RB, hd), lambda i: (i, 0)),
            wspec, wspec, bspec, wspec, bspec,
        ],
        out_specs=pl.BlockSpec((_RB, hd), lambda i: (i, 0)),
        out_shape=jax.ShapeDtypeStruct((n, hd), jnp.float32),
    )(h_init, h, w1a, w1b, b1, w2, b2)


_CHUNK = 128


@functools.lru_cache(maxsize=None)
def _make_sc_agg(n_nodes, h_dim, e_pad, npad, nc, ns):
    nw = nc * ns
    per_w = e_pad // nw
    n_chunks = per_w // _CHUNK
    zrows = npad // ns

    mesh = plsc.VectorSubcoreMesh(core_axis_name="c", subcore_axis_name="s")

    @functools.partial(
        pl.kernel,
        out_type=jax.ShapeDtypeStruct((nc, npad, h_dim), jnp.float32),
        mesh=mesh,
        scratch_types=[
            pltpu.VMEM((_CHUNK,), jnp.int32),
            pltpu.VMEM((_CHUNK,), jnp.int32),
            pltpu.VMEM((_CHUNK, h_dim), jnp.float32),
            pltpu.VMEM((16, h_dim), jnp.float32),
            pltpu.VMEM_SHARED((npad, h_dim), jnp.float32),
            pltpu.SemaphoreType.DMA,
        ],
        compiler_params=pltpu.CompilerParams(use_tc_tiling_on_sc=False),
    )
    def sc_agg(table_hbm, fidx_hbm, dst_hbm, out_hbm,
               idx_v, dst_v, rows_v, zero_v, acc_sh, sem):
        cid = lax.axis_index("c")
        sid = lax.axis_index("s")
        wid = sid * nc + cid

        for i in range(16):
            for j in range(h_dim // 16):
                zero_v[i, pl.ds(j * 16, 16)] = jnp.zeros((16,), jnp.float32)

        def zbody(k, c):
            pltpu.sync_copy(zero_v, acc_sh.at[pl.ds(sid * zrows + k * 16, 16)])
            return c

        lax.fori_loop(0, zrows // 16, zbody, 0)
        plsc.subcore_barrier()

        base0 = wid * per_w

        def body(i, c):
            base = base0 + i * _CHUNK
            pltpu.sync_copy(fidx_hbm.at[pl.ds(base, _CHUNK)], idx_v)
            pltpu.sync_copy(dst_hbm.at[pl.ds(base, _CHUNK)], dst_v)
            pltpu.async_copy(table_hbm.at[idx_v], rows_v, sem).wait()
            pltpu.sync_copy(rows_v, acc_sh.at[dst_v], add=True)
            return c

        lax.fori_loop(0, n_chunks, body, 0)
        plsc.subcore_barrier()

        pltpu.sync_copy(
            acc_sh.at[pl.ds(sid * zrows, zrows)],
            out_hbm.at[cid, pl.ds(sid * zrows, zrows)],
        )

    return sc_agg


def kernel(x, edge_index, edge_attr, W_in, b_in, edge_W, edge_b,
           gru_W_ih, gru_W_hh, gru_b_ih, gru_b_hh, W_t1, b_t1, W_t2, b_t2):
    n, _ = x.shape
    e = edge_index.shape[1]
    t, h, _ = edge_W.shape

    info = plsc.get_sparse_core_info()
    nc, ns = info.num_cores, info.num_subcores
    nw = nc * ns

    et = jnp.argmax(edge_attr, axis=1).astype(jnp.int32)
    fidx = et * n + edge_index[0]
    dst = edge_index[1]

    quant = nw * _CHUNK
    e_pad = ((e + quant - 1) // quant) * quant
    npad = ((n + 16 * ns) // (16 * ns)) * (16 * ns)
    if e_pad > e:
        fidx = jnp.concatenate([fidx, jnp.zeros((e_pad - e,), jnp.int32)])
        dst = jnp.concatenate(
            [dst, jnp.full((e_pad - e,), npad - 1, jnp.int32)])

    sc_agg = _make_sc_agg(n, h, e_pad, npad, nc, ns)

    wir, wiz, win = (gru_W_ih[:h].T, gru_W_ih[h:2 * h].T, gru_W_ih[2 * h:].T)
    whr, whz, whn = (gru_W_hh[:h].T, gru_W_hh[h:2 * h].T, gru_W_hh[2 * h:].T)
    br = (gru_b_ih[:h] + gru_b_hh[:h]).reshape(1, h)
    bz = (gru_b_ih[h:2 * h] + gru_b_hh[h:2 * h]).reshape(1, h)
    bni = gru_b_ih[2 * h:].reshape(1, h)
    bnh = gru_b_hh[2 * h:].reshape(1, h)
    gws = (wir, wiz, win, whr, whz, whn)
    gbs = (br, bz, bni, bnh)

    h_init = _hinit(x, W_in, b_in.reshape(1, h))
    hs = h_init
    for _ in range(_STEPS):
        m = _mtable(hs, edge_W, edge_b).reshape(t * n, h)
        agg2 = sc_agg(m, fidx, dst)[:, :n]
        hs = _gru(agg2, hs, gws, gbs)

    return _final(h_init, hs, W_t1[:h], W_t1[h:], b_t1.reshape(1, h),
                  W_t2, b_t2.reshape(1, h))
